# baseline (speedup 1.0000x reference)
import functools

import jax
import jax.numpy as jnp
from jax import lax
from jax.experimental import pallas as pl
from jax.experimental.pallas import tpu as pltpu
from jax.experimental.pallas import tpu_sc as plsc

N = 10000
E = 320000
D = 128
C = 1.0
MIN_NORM = 1e-15
BALL_EPS = 4e-3

NC = 2
NS = 16
NW = NC * NS
CHUNK = 128
NCH = E // CHUNK
CH_PER_W = NCH // NW
CH_REM = NCH - CH_PER_W * NW
NBUF = 2
TILE_STRIDE = 624
TILE_SPAN = 640

TC_BLOCK = 1000


def _artanh(z):
    z = jnp.clip(z, -1.0 + 1e-7, 1.0 - 1e-7)
    return 0.5 * jnp.log((1.0 + z) / (1.0 - z))


def _logmap0_body(x_ref, o_ref):
    v = x_ref[...]
    nrm = jnp.sqrt(jnp.sum(v * v, axis=1, keepdims=True))
    nrm = jnp.maximum(nrm, MIN_NORM)
    scale = _artanh(nrm) / nrm
    o_ref[...] = scale * v


def _epilogue_body(a_ref, b_ref, o_ref):
    s = a_ref[0] + b_ref[0]
    u_nrm = jnp.maximum(
        jnp.sqrt(jnp.sum(s * s, axis=1, keepdims=True)), MIN_NORM)
    y = jnp.tanh(u_nrm) * s / u_nrm
    y_nrm = jnp.maximum(
        jnp.sqrt(jnp.sum(y * y, axis=1, keepdims=True)), MIN_NORM)
    maxnorm = 1.0 - BALL_EPS
    o_ref[...] = jnp.where(y_nrm > maxnorm, y / y_nrm * maxnorm, y)


def _spmm_body(xt_hbm, pk_hbm, adj_hbm, out_hbm,
               pk_0, pk_1, pk_2, pk_3, pk_4, pk_5,
               adj_0, adj_1,
               rows_0, rows_1, rows_2, acc,
               g_0, g_1, g_2, s_0, s_1, s_2,
               t_0, t_1, t_2, t_3, t_4, t_5, a_0, a_1):
    cid = lax.axis_index("c")
    sid = lax.axis_index("s")
    wid = cid * NS + sid

    pks = [pk_0, pk_1, pk_2, pk_3, pk_4, pk_5]
    adjs = [adj_0, adj_1]
    rows = [rows_0, rows_1, rows_2]
    gsems = [g_0, g_1, g_2]
    ssems = [s_0, s_1, s_2]
    tsems = [t_0, t_1, t_2, t_3, t_4, t_5]
    asems = [a_0, a_1]

    def zfill(e, carry):
        z = jnp.zeros((16,), jnp.float32)
        for j in range(D // 16):
            rows_0[e, pl.ds(j * 16, 16)] = z
        return carry
    lax.fori_loop(0, CHUNK, zfill, 0, unroll=False)
    for k in range(TILE_SPAN // CHUNK):
        pltpu.sync_copy(
            rows_0, acc.at[pl.ds(sid * TILE_STRIDE + k * CHUNK, CHUNK)])
    plsc.subcore_barrier()

    def cix(i):
        return jnp.minimum(wid + i * NW, NCH - 1)

    def tstart(b, i):
        pltpu.async_copy(pk_hbm.at[cix(i)], pks[b], tsems[b])

    def twait(b):
        pltpu.make_async_copy(pk_hbm.at[0], pks[b], tsems[b]).wait()

    def astart(a, i):
        pltpu.async_copy(adj_hbm.at[cix(i)], adjs[a], asems[a])

    def await_adj(a):
        pltpu.make_async_copy(adj_hbm.at[0], adjs[a], asems[a]).wait()

    def gstart(b, q):
        twait(q)
        pltpu.async_copy(xt_hbm.at[pks[q].at[0]], rows[b], gsems[b])

    def gwait(b, q):
        pltpu.make_async_copy(xt_hbm.at[pks[q].at[0]], rows[b],
                              gsems[b]).wait()

    def scale(a, b):
        adj_v, rows_v = adjs[a], rows[b]

        def scale_body(g, carry):
            off = pl.multiple_of(g * 16, 16)
            s16 = adj_v[0, pl.ds(off, 16)]
            for l in range(16):
                e = off + l
                s = s16[l]
                for j in range(D // 16):
                    sl = rows_v[e, pl.ds(j * 16, 16)]
                    rows_v[e, pl.ds(j * 16, 16)] = sl * s
            return carry
        lax.fori_loop(0, CHUNK // 16, scale_body, 0, unroll=False)

    def sstart(b, q):
        pltpu.async_copy(rows[b], acc.at[pks[q].at[1]], ssems[b], add=True)

    def swait(b, q):
        pltpu.make_async_copy(rows[b], acc.at[pks[q].at[1]],
                              ssems[b]).wait()

    def emit_pos(j, i, skip_swait=False):
        b = j % 3
        a = j % 2
        gwait(b, j % 6)
        await_adj(a)
        scale(a, b)
        astart(a, i + 2)
        sstart(b, j % 6)
        if not skip_swait:
            swait((b + 1) % 3, (j + 4) % 6)
        tstart((j + 4) % 6, i + 4)
        gstart((b + 1) % 3, (j + 1) % 6)

    for b in range(4):
        tstart(b, b)
    astart(0, 0)
    astart(1, 1)
    gstart(0, 0)
    for j in range(6):
        emit_pos(j, j, skip_swait=(j < 2))

    def body(k, carry):
        i0 = 6 * k
        for j in range(6):
            emit_pos(j, i0 + j)
        return carry

    lax.fori_loop(1, CH_PER_W // 6, body, 0, unroll=False)

    swait(1, 2)
    swait(2, 3)
    gwait(0, 0)
    await_adj(0)
    scale(0, 0)

    @pl.when(wid < CH_REM)
    def _():
        pltpu.sync_copy(rows[0], acc.at[pks[0].at[1]], add=True)

    twait(1)
    twait(2)
    twait(3)
    await_adj(1)

    plsc.subcore_barrier()

    pltpu.sync_copy(acc.at[pl.ds(sid * TILE_STRIDE, TILE_SPAN)],
                    out_hbm.at[cid, pl.ds(sid * TILE_STRIDE, TILE_SPAN)])


_spmm = functools.partial(
    pl.kernel,
    out_type=jax.ShapeDtypeStruct((NC, N, D), jnp.float32),
    mesh=plsc.VectorSubcoreMesh(core_axis_name="c", subcore_axis_name="s",
                                num_cores=NC, num_subcores=NS),
    scratch_types=(
        [pltpu.VMEM((2, CHUNK), jnp.int32) for _ in range(6)] +
        [pltpu.VMEM((1, CHUNK), jnp.float32) for _ in range(2)] +
        [pltpu.VMEM((CHUNK, D), jnp.float32) for _ in range(3)] +
        [pltpu.VMEM_SHARED((N, D), jnp.float32)] +
        [pltpu.SemaphoreType.DMA for _ in range(14)]
    ),
)(_spmm_body)


def kernel(x, edge_index, adj_values):
    x = x.astype(jnp.float32)
    src = edge_index[0].astype(jnp.int32)
    dst = edge_index[1].astype(jnp.int32)
    adj3d = adj_values.astype(jnp.float32).reshape(NCH, 1, CHUNK)
    packed = jnp.stack([src.reshape(NCH, CHUNK),
                        dst.reshape(NCH, CHUNK)], axis=1)

    xt = pl.pallas_call(
        _logmap0_body,
        out_shape=jax.ShapeDtypeStruct((N, D), jnp.float32),
        grid=(N // TC_BLOCK,),
        in_specs=[pl.BlockSpec((TC_BLOCK, D), lambda i: (i, 0))],
        out_specs=pl.BlockSpec((TC_BLOCK, D), lambda i: (i, 0)),
    )(x)

    partials = _spmm(xt, packed, adj3d)

    out = pl.pallas_call(
        _epilogue_body,
        out_shape=jax.ShapeDtypeStruct((N, D), jnp.float32),
        grid=(N // TC_BLOCK,),
        in_specs=[pl.BlockSpec((1, TC_BLOCK, D), lambda i: (0, i, 0)),
                  pl.BlockSpec((1, TC_BLOCK, D), lambda i: (1, i, 0))],
        out_specs=pl.BlockSpec((TC_BLOCK, D), lambda i: (i, 0)),
    )(partials, partials)
    return out


# speedup vs baseline: 1.2152x; 1.2152x over previous
import functools

import jax
import jax.numpy as jnp
from jax import lax
from jax.experimental import pallas as pl
from jax.experimental.pallas import tpu as pltpu
from jax.experimental.pallas import tpu_sc as plsc

N = 10000
E = 320000
D = 128
C = 1.0
MIN_NORM = 1e-15
BALL_EPS = 4e-3

NC = 2
NS = 16
NW = NC * NS
CHUNK = 128
NCH = E // CHUNK
CH_PER_W = NCH // NW
CH_REM = NCH - CH_PER_W * NW
NBUF = 2
TILE_STRIDE = 624
TILE_SPAN = 640

TC_BLOCK = 1000


def _artanh(z):
    z = jnp.clip(z, -1.0 + 1e-7, 1.0 - 1e-7)
    return 0.5 * jnp.log((1.0 + z) / (1.0 - z))


def _logmap0_body(x_ref, o_ref):
    v = x_ref[...]
    nrm = jnp.sqrt(jnp.sum(v * v, axis=1, keepdims=True))
    nrm = jnp.maximum(nrm, MIN_NORM)
    scale = _artanh(nrm) / nrm
    o_ref[...] = scale * v


def _epilogue_body(a_ref, b_ref, o_ref):
    s = a_ref[0] + b_ref[0]
    u_nrm = jnp.maximum(
        jnp.sqrt(jnp.sum(s * s, axis=1, keepdims=True)), MIN_NORM)
    y = jnp.tanh(u_nrm) * s / u_nrm
    y_nrm = jnp.maximum(
        jnp.sqrt(jnp.sum(y * y, axis=1, keepdims=True)), MIN_NORM)
    maxnorm = 1.0 - BALL_EPS
    o_ref[...] = jnp.where(y_nrm > maxnorm, y / y_nrm * maxnorm, y)


def _spmm_body(xt_hbm, pk_hbm, adj_hbm, out_hbm,
               pk_0, pk_1, pk_2, adj_0, adj_1, adj_2,
               rows_0, rows_1, rows_2, acc,
               g_0, g_1, g_2, s_0, s_1, s_2, t_0, t_1, t_2):
    cid = lax.axis_index("c")
    sid = lax.axis_index("s")
    wid = cid * NS + sid

    pks = [pk_0, pk_1, pk_2]
    adjs = [adj_0, adj_1, adj_2]
    rows = [rows_0, rows_1, rows_2]
    gsems = [g_0, g_1, g_2]
    ssems = [s_0, s_1, s_2]
    tsems = [t_0, t_1, t_2]

    def zfill(e, carry):
        z = jnp.zeros((16,), jnp.float32)
        for j in range(D // 16):
            rows_0[e, pl.ds(j * 16, 16)] = z
        return carry
    lax.fori_loop(0, CHUNK, zfill, 0, unroll=False)
    for k in range(TILE_SPAN // CHUNK):
        pltpu.sync_copy(
            rows_0, acc.at[pl.ds(sid * TILE_STRIDE + k * CHUNK, CHUNK)])
    plsc.subcore_barrier()

    def cix(i):
        return jnp.minimum(wid + i * NW, NCH - 1)

    def tstart(b, i):
        c = cix(i)
        pltpu.async_copy(pk_hbm.at[c], pks[b], tsems[b])
        pltpu.async_copy(adj_hbm.at[c], adjs[b], tsems[b])

    def twait(b):
        pltpu.make_async_copy(pk_hbm.at[0], pks[b], tsems[b]).wait()
        pltpu.make_async_copy(adj_hbm.at[0], adjs[b], tsems[b]).wait()

    def gstart(b):
        twait(b)
        pltpu.async_copy(xt_hbm.at[pks[b].at[0]], rows[b], gsems[b])

    def gwait(b):
        pltpu.make_async_copy(xt_hbm.at[pks[b].at[0]], rows[b],
                              gsems[b]).wait()

    def scale(b):
        adj_v, rows_v = adjs[b], rows[b]

        def scale_body(g, carry):
            off = pl.multiple_of(g * 16, 16)
            s16 = adj_v[0, pl.ds(off, 16)]
            for l in range(16):
                e = off + l
                s = s16[l]
                for j in range(D // 16):
                    sl = rows_v[e, pl.ds(j * 16, 16)]
                    rows_v[e, pl.ds(j * 16, 16)] = sl * s
            return carry
        lax.fori_loop(0, CHUNK // 16, scale_body, 0, unroll=False)

    def sstart(b):
        pltpu.async_copy(rows[b], acc.at[pks[b].at[1]], ssems[b], add=True)

    def swait(b):
        pltpu.make_async_copy(rows[b], acc.at[pks[b].at[1]],
                              ssems[b]).wait()

    for b in range(3):
        tstart(b, b)
    gstart(0)
    gstart(1)

    def body(k, carry):
        i0 = 3 * k
        gstart(2)
        gwait(0)
        scale(0)
        sstart(0)
        gwait(1)
        scale(1)
        sstart(1)
        swait(0)
        tstart(0, i0 + 3)
        gstart(0)
        gwait(2)
        scale(2)
        sstart(2)
        swait(1)
        tstart(1, i0 + 4)
        gstart(1)
        swait(2)
        tstart(2, i0 + 5)
        return carry

    lax.fori_loop(0, CH_PER_W // 3, body, 0, unroll=False)

    gwait(0)
    scale(0)

    @pl.when(wid < CH_REM)
    def _():
        pltpu.sync_copy(rows[0], acc.at[pks[0].at[1]], add=True)

    gwait(1)
    twait(2)

    plsc.subcore_barrier()

    pltpu.sync_copy(acc.at[pl.ds(sid * TILE_STRIDE, TILE_SPAN)],
                    out_hbm.at[cid, pl.ds(sid * TILE_STRIDE, TILE_SPAN)])


_spmm = functools.partial(
    pl.kernel,
    out_type=jax.ShapeDtypeStruct((NC, N, D), jnp.float32),
    mesh=plsc.VectorSubcoreMesh(core_axis_name="c", subcore_axis_name="s",
                                num_cores=NC, num_subcores=NS),
    scratch_types=(
        [pltpu.VMEM((2, CHUNK), jnp.int32) for _ in range(3)] +
        [pltpu.VMEM((1, CHUNK), jnp.float32) for _ in range(3)] +
        [pltpu.VMEM((CHUNK, D), jnp.float32) for _ in range(3)] +
        [pltpu.VMEM_SHARED((N, D), jnp.float32)] +
        [pltpu.SemaphoreType.DMA for _ in range(9)]
    ),
)(_spmm_body)


def kernel(x, edge_index, adj_values):
    x = x.astype(jnp.float32)
    src = edge_index[0].astype(jnp.int32)
    dst = edge_index[1].astype(jnp.int32)
    adj3d = adj_values.astype(jnp.float32).reshape(NCH, 1, CHUNK)
    packed = jnp.stack([src.reshape(NCH, CHUNK),
                        dst.reshape(NCH, CHUNK)], axis=1)

    xt = pl.pallas_call(
        _logmap0_body,
        out_shape=jax.ShapeDtypeStruct((N, D), jnp.float32),
        grid=(N // TC_BLOCK,),
        in_specs=[pl.BlockSpec((TC_BLOCK, D), lambda i: (i, 0))],
        out_specs=pl.BlockSpec((TC_BLOCK, D), lambda i: (i, 0)),
    )(x)

    partials = _spmm(xt, packed, adj3d)

    out = pl.pallas_call(
        _epilogue_body,
        out_shape=jax.ShapeDtypeStruct((N, D), jnp.float32),
        grid=(N // TC_BLOCK,),
        in_specs=[pl.BlockSpec((1, TC_BLOCK, D), lambda i: (0, i, 0)),
                  pl.BlockSpec((1, TC_BLOCK, D), lambda i: (1, i, 0))],
        out_specs=pl.BlockSpec((TC_BLOCK, D), lambda i: (i, 0)),
    )(partials, partials)
    return out


# speedup vs baseline: 1.2230x; 1.0065x over previous
import functools

import jax
import jax.numpy as jnp
from jax import lax
from jax.experimental import pallas as pl
from jax.experimental.pallas import tpu as pltpu
from jax.experimental.pallas import tpu_sc as plsc

N = 10000
E = 320000
D = 128
C = 1.0
MIN_NORM = 1e-15
BALL_EPS = 4e-3

NC = 2
NS = 16
NW = NC * NS
CHUNK = 128
NCH = E // CHUNK
CH_PER_W = NCH // NW
CH_REM = NCH - CH_PER_W * NW
NBUF = 2
TILE_STRIDE = 624
TILE_SPAN = 640

TC_BLOCK = 1000


def _artanh(z):
    z = jnp.clip(z, -1.0 + 1e-7, 1.0 - 1e-7)
    return 0.5 * jnp.log((1.0 + z) / (1.0 - z))


def _logmap0_body(x_ref, o_ref):
    v = x_ref[...]
    nrm = jnp.sqrt(jnp.sum(v * v, axis=1, keepdims=True))
    nrm = jnp.maximum(nrm, MIN_NORM)
    scale = _artanh(nrm) / nrm
    o_ref[...] = scale * v


def _epilogue_body(a_ref, b_ref, o_ref):
    s = a_ref[0] + b_ref[0]
    u_nrm = jnp.maximum(
        jnp.sqrt(jnp.sum(s * s, axis=1, keepdims=True)), MIN_NORM)
    y = jnp.tanh(u_nrm) * s / u_nrm
    y_nrm = jnp.maximum(
        jnp.sqrt(jnp.sum(y * y, axis=1, keepdims=True)), MIN_NORM)
    maxnorm = 1.0 - BALL_EPS
    o_ref[...] = jnp.where(y_nrm > maxnorm, y / y_nrm * maxnorm, y)


def _spmm_body(xt_hbm, pk_hbm, adj_hbm, out_hbm,
               pk_0, pk_1, pk_2, adj_0, adj_1, adj_2,
               rows_0, rows_1, rows_2, acc,
               g_0, g_1, g_2, s_0, s_1, s_2, t_0, t_1, t_2):
    cid = lax.axis_index("c")
    sid = lax.axis_index("s")
    wid = cid * NS + sid

    pks = [pk_0, pk_1, pk_2]
    adjs = [adj_0, adj_1, adj_2]
    rows = [rows_0, rows_1, rows_2]
    gsems = [g_0, g_1, g_2]
    ssems = [s_0, s_1, s_2]
    tsems = [t_0, t_1, t_2]

    def zfill(e, carry):
        z = jnp.zeros((16,), jnp.float32)
        for j in range(D // 16):
            rows_0[e, pl.ds(j * 16, 16)] = z
        return carry
    lax.fori_loop(0, CHUNK, zfill, 0, unroll=False)
    for k in range(TILE_SPAN // CHUNK):
        pltpu.sync_copy(
            rows_0, acc.at[pl.ds(sid * TILE_STRIDE + k * CHUNK, CHUNK)])
    plsc.subcore_barrier()

    def cix(i):
        return jnp.minimum(wid + i * NW, NCH - 1)

    def tstart(b, i):
        c = cix(i)
        pltpu.async_copy(pk_hbm.at[c], pks[b], tsems[b])
        pltpu.async_copy(adj_hbm.at[c], adjs[b], tsems[b])

    def twait(b):
        pltpu.make_async_copy(pk_hbm.at[0], pks[b], tsems[b]).wait()
        pltpu.make_async_copy(adj_hbm.at[0], adjs[b], tsems[b]).wait()

    def gstart(b):
        twait(b)
        pltpu.async_copy(xt_hbm.at[pks[b].at[0]], rows[b], gsems[b])

    def gwait(b):
        pltpu.make_async_copy(xt_hbm.at[pks[b].at[0]], rows[b],
                              gsems[b]).wait()

    def scale(b):
        adj_v, rows_v = adjs[b], rows[b]

        def scale_body(g, carry):
            off = pl.multiple_of(g * 16, 16)
            s16 = adj_v[0, pl.ds(off, 16)]
            for l in range(16):
                e = off + l
                s = s16[l]
                for j in range(D // 16):
                    sl = rows_v[e, pl.ds(j * 16, 16)]
                    rows_v[e, pl.ds(j * 16, 16)] = sl * s
            return carry
        lax.fori_loop(0, CHUNK // 16, scale_body, 0, unroll=False)

    def sstart(b):
        pltpu.async_copy(rows[b], acc.at[pks[b].at[1]], ssems[b], add=True)

    def swait(b):
        pltpu.make_async_copy(rows[b], acc.at[pks[b].at[1]],
                              ssems[b]).wait()

    for b in range(3):
        tstart(b, b)
    gstart(0)
    gstart(1)

    def body(k, carry):
        i0 = 3 * k
        gstart(2)
        gwait(0)
        scale(0)
        sstart(0)
        swait(0)
        tstart(0, i0 + 3)
        gwait(1)
        scale(1)
        sstart(1)
        gstart(0)
        swait(1)
        tstart(1, i0 + 4)
        gwait(2)
        scale(2)
        sstart(2)
        gstart(1)
        swait(2)
        tstart(2, i0 + 5)
        return carry

    lax.fori_loop(0, CH_PER_W // 3, body, 0, unroll=False)

    gwait(0)
    scale(0)

    @pl.when(wid < CH_REM)
    def _():
        pltpu.sync_copy(rows[0], acc.at[pks[0].at[1]], add=True)

    gwait(1)
    twait(2)

    plsc.subcore_barrier()

    pltpu.sync_copy(acc.at[pl.ds(sid * TILE_STRIDE, TILE_SPAN)],
                    out_hbm.at[cid, pl.ds(sid * TILE_STRIDE, TILE_SPAN)])


_spmm = functools.partial(
    pl.kernel,
    out_type=jax.ShapeDtypeStruct((NC, N, D), jnp.float32),
    mesh=plsc.VectorSubcoreMesh(core_axis_name="c", subcore_axis_name="s",
                                num_cores=NC, num_subcores=NS),
    scratch_types=(
        [pltpu.VMEM((2, CHUNK), jnp.int32) for _ in range(3)] +
        [pltpu.VMEM((1, CHUNK), jnp.float32) for _ in range(3)] +
        [pltpu.VMEM((CHUNK, D), jnp.float32) for _ in range(3)] +
        [pltpu.VMEM_SHARED((N, D), jnp.float32)] +
        [pltpu.SemaphoreType.DMA for _ in range(9)]
    ),
)(_spmm_body)


def kernel(x, edge_index, adj_values):
    x = x.astype(jnp.float32)
    src = edge_index[0].astype(jnp.int32)
    dst = edge_index[1].astype(jnp.int32)
    adj3d = adj_values.astype(jnp.float32).reshape(NCH, 1, CHUNK)
    packed = jnp.stack([src.reshape(NCH, CHUNK),
                        dst.reshape(NCH, CHUNK)], axis=1)

    xt = pl.pallas_call(
        _logmap0_body,
        out_shape=jax.ShapeDtypeStruct((N, D), jnp.float32),
        grid=(N // TC_BLOCK,),
        in_specs=[pl.BlockSpec((TC_BLOCK, D), lambda i: (i, 0))],
        out_specs=pl.BlockSpec((TC_BLOCK, D), lambda i: (i, 0)),
    )(x)

    partials = _spmm(xt, packed, adj3d)

    out = pl.pallas_call(
        _epilogue_body,
        out_shape=jax.ShapeDtypeStruct((N, D), jnp.float32),
        grid=(N // TC_BLOCK,),
        in_specs=[pl.BlockSpec((1, TC_BLOCK, D), lambda i: (0, i, 0)),
                  pl.BlockSpec((1, TC_BLOCK, D), lambda i: (1, i, 0))],
        out_specs=pl.BlockSpec((TC_BLOCK, D), lambda i: (i, 0)),
    )(partials, partials)
    return out


# speedup vs baseline: 1.2413x; 1.0149x over previous
import functools

import jax
import jax.numpy as jnp
from jax import lax
from jax.experimental import pallas as pl
from jax.experimental.pallas import tpu as pltpu
from jax.experimental.pallas import tpu_sc as plsc

N = 10000
E = 320000
D = 128
C = 1.0
MIN_NORM = 1e-15
BALL_EPS = 4e-3

NC = 2
NS = 16
NW = NC * NS
CHUNK = 128
NCH = E // CHUNK
CH_PER_W = NCH // NW
CH_REM = NCH - CH_PER_W * NW
NBUF = 2
TILE_STRIDE = 624
TILE_SPAN = 640

TC_BLOCK = 2000


def _artanh(z):
    z = jnp.clip(z, -1.0 + 1e-7, 1.0 - 1e-7)
    return 0.5 * jnp.log((1.0 + z) / (1.0 - z))


def _logmap0_body(x_ref, o_ref):
    v = x_ref[...]
    nrm = jnp.sqrt(jnp.sum(v * v, axis=1, keepdims=True))
    nrm = jnp.maximum(nrm, MIN_NORM)
    scale = _artanh(nrm) / nrm
    o_ref[...] = scale * v


def _epilogue_body(a_ref, b_ref, o_ref):
    s = a_ref[0] + b_ref[0]
    u_nrm = jnp.maximum(
        jnp.sqrt(jnp.sum(s * s, axis=1, keepdims=True)), MIN_NORM)
    y = jnp.tanh(u_nrm) * s / u_nrm
    y_nrm = jnp.maximum(
        jnp.sqrt(jnp.sum(y * y, axis=1, keepdims=True)), MIN_NORM)
    maxnorm = 1.0 - BALL_EPS
    o_ref[...] = jnp.where(y_nrm > maxnorm, y / y_nrm * maxnorm, y)


def _spmm_body(xt_hbm, pk_hbm, adj_hbm, out_hbm,
               pk_0, pk_1, pk_2, adj_0, adj_1, adj_2,
               rows_0, rows_1, rows_2, acc,
               g_0, g_1, g_2, s_0, s_1, s_2, t_0, t_1, t_2):
    cid = lax.axis_index("c")
    sid = lax.axis_index("s")
    wid = cid * NS + sid

    pks = [pk_0, pk_1, pk_2]
    adjs = [adj_0, adj_1, adj_2]
    rows = [rows_0, rows_1, rows_2]
    gsems = [g_0, g_1, g_2]
    ssems = [s_0, s_1, s_2]
    tsems = [t_0, t_1, t_2]

    def zfill(e, carry):
        z = jnp.zeros((16,), jnp.float32)
        for j in range(D // 16):
            rows_0[e, pl.ds(j * 16, 16)] = z
        return carry
    lax.fori_loop(0, CHUNK, zfill, 0, unroll=False)
    for k in range(TILE_SPAN // CHUNK):
        pltpu.sync_copy(
            rows_0, acc.at[pl.ds(sid * TILE_STRIDE + k * CHUNK, CHUNK)])
    plsc.subcore_barrier()

    def cix(i):
        return jnp.minimum(wid + i * NW, NCH - 1)

    def tstart(b, i):
        c = cix(i)
        pltpu.async_copy(pk_hbm.at[c], pks[b], tsems[b])
        pltpu.async_copy(adj_hbm.at[c], adjs[b], tsems[b])

    def twait(b):
        pltpu.make_async_copy(pk_hbm.at[0], pks[b], tsems[b]).wait()
        pltpu.make_async_copy(adj_hbm.at[0], adjs[b], tsems[b]).wait()

    def gstart(b):
        twait(b)
        pltpu.async_copy(xt_hbm.at[pks[b].at[0]], rows[b], gsems[b])

    def gwait(b):
        pltpu.make_async_copy(xt_hbm.at[pks[b].at[0]], rows[b],
                              gsems[b]).wait()

    def scale(b):
        adj_v, rows_v = adjs[b], rows[b]

        def scale_body(g, carry):
            off = pl.multiple_of(g * 16, 16)
            s16 = adj_v[0, pl.ds(off, 16)]
            for l in range(16):
                e = off + l
                s = s16[l]
                for j in range(D // 16):
                    sl = rows_v[e, pl.ds(j * 16, 16)]
                    rows_v[e, pl.ds(j * 16, 16)] = sl * s
            return carry
        lax.fori_loop(0, CHUNK // 16, scale_body, 0, unroll=2)

    def sstart(b):
        pltpu.async_copy(rows[b], acc.at[pks[b].at[1]], ssems[b], add=True)

    def swait(b):
        pltpu.make_async_copy(rows[b], acc.at[pks[b].at[1]],
                              ssems[b]).wait()

    for b in range(3):
        tstart(b, b)
    gstart(0)
    gstart(1)

    def body(k, carry):
        i0 = 3 * k
        gstart(2)
        gwait(0)
        scale(0)
        sstart(0)
        swait(0)
        tstart(0, i0 + 3)
        gwait(1)
        scale(1)
        sstart(1)
        gstart(0)
        swait(1)
        tstart(1, i0 + 4)
        gwait(2)
        scale(2)
        sstart(2)
        gstart(1)
        swait(2)
        tstart(2, i0 + 5)
        return carry

    lax.fori_loop(0, CH_PER_W // 3, body, 0, unroll=False)

    gwait(0)
    scale(0)

    @pl.when(wid < CH_REM)
    def _():
        pltpu.sync_copy(rows[0], acc.at[pks[0].at[1]], add=True)

    gwait(1)
    twait(2)

    plsc.subcore_barrier()

    pltpu.sync_copy(acc.at[pl.ds(sid * TILE_STRIDE, TILE_SPAN)],
                    out_hbm.at[cid, pl.ds(sid * TILE_STRIDE, TILE_SPAN)])


_spmm = functools.partial(
    pl.kernel,
    out_type=jax.ShapeDtypeStruct((NC, N, D), jnp.float32),
    mesh=plsc.VectorSubcoreMesh(core_axis_name="c", subcore_axis_name="s",
                                num_cores=NC, num_subcores=NS),
    scratch_types=(
        [pltpu.VMEM((2, CHUNK), jnp.int32) for _ in range(3)] +
        [pltpu.VMEM((1, CHUNK), jnp.float32) for _ in range(3)] +
        [pltpu.VMEM((CHUNK, D), jnp.float32) for _ in range(3)] +
        [pltpu.VMEM_SHARED((N, D), jnp.float32)] +
        [pltpu.SemaphoreType.DMA for _ in range(9)]
    ),
)(_spmm_body)


def kernel(x, edge_index, adj_values):
    x = x.astype(jnp.float32)
    src = edge_index[0].astype(jnp.int32)
    dst = edge_index[1].astype(jnp.int32)
    adj3d = adj_values.astype(jnp.float32).reshape(NCH, 1, CHUNK)
    packed = jnp.stack([src.reshape(NCH, CHUNK),
                        dst.reshape(NCH, CHUNK)], axis=1)

    xt = pl.pallas_call(
        _logmap0_body,
        out_shape=jax.ShapeDtypeStruct((N, D), jnp.float32),
        grid=(N // TC_BLOCK,),
        in_specs=[pl.BlockSpec((TC_BLOCK, D), lambda i: (i, 0))],
        out_specs=pl.BlockSpec((TC_BLOCK, D), lambda i: (i, 0)),
    )(x)

    partials = _spmm(xt, packed, adj3d)

    out = pl.pallas_call(
        _epilogue_body,
        out_shape=jax.ShapeDtypeStruct((N, D), jnp.float32),
        grid=(N // TC_BLOCK,),
        in_specs=[pl.BlockSpec((1, TC_BLOCK, D), lambda i: (0, i, 0)),
                  pl.BlockSpec((1, TC_BLOCK, D), lambda i: (1, i, 0))],
        out_specs=pl.BlockSpec((TC_BLOCK, D), lambda i: (i, 0)),
    )(partials, partials)
    return out
